# baseline (speedup 1.0000x reference)
import jax
import jax.numpy as jnp
from jax import lax
from jax.experimental import pallas as pl
from jax.experimental.pallas import tpu as pltpu
from jax.experimental.pallas import tpu_sc as plsc

NC, NS = 2, 16
NW = NC * NS
CH = 80
DCH = 80
NBUF = 3
ZR = 80
NSPLIT = 1
DH = 128 // NSPLIT

_MESH = dict(core_axis_name="c", subcore_axis_name="s")


def _deg_body(dst_hbm, out_hbm, dst_v, ones_v, zbuf_v, deg_sh, dsem):
    c = lax.axis_index("c")
    s = lax.axis_index("s")
    wid = s * NC + c
    n = deg_sh.shape[0]
    pltpu.sync_copy(dst_hbm.at[wid], dst_v)

    def fill_ones(i, _):
        ones_v[pl.ds(i * 16, 16)] = jnp.ones((16,), jnp.float32)
        return 0

    lax.fori_loop(0, DCH // 16, fill_ones, 0)

    def fill_zeros(i, _):
        zbuf_v[pl.ds(i * 16, 16)] = jnp.zeros((16,), jnp.float32)
        return 0

    lax.fori_loop(0, n // 16, fill_zeros, 0)

    @pl.when(s == 0)
    def _():
        pltpu.sync_copy(zbuf_v, deg_sh)

    plsc.subcore_barrier()

    nchunks = dst_v.shape[0]
    K = 5

    def body(g, _):
        for u in range(K):
            pltpu.async_copy(ones_v, deg_sh.at[dst_v.at[g * K + u]],
                             dsem, add=True)
        for u in range(K):
            pltpu.make_async_copy(
                ones_v, deg_sh.at[dst_v.at[g * K + u]], dsem).wait()
        return 0

    lax.fori_loop(0, nchunks // K, body, 0)
    for j in range((nchunks // K) * K, nchunks):
        pltpu.sync_copy(ones_v, deg_sh.at[dst_v.at[j]], add=True)
    plsc.subcore_barrier()

    @pl.when(s == 0)
    def _():
        pltpu.sync_copy(deg_sh, zbuf_v)
        pltpu.sync_copy(zbuf_v, out_hbm.at[pl.ds(c * n, n)])


def _agg_body(src_hbm, dst_hbm, *refs):
    h2_hbms = refs[:NSPLIT]
    out_hbm = refs[NSPLIT]
    src_v = refs[NSPLIT + 1]
    dst_v = refs[NSPLIT + 2]
    rows = refs[NSPLIT + 3:NSPLIT + 3 + NBUF]
    acc_sh = refs[NSPLIT + 3 + NBUF]
    sems = refs[NSPLIT + 4 + NBUF:NSPLIT + 4 + 2 * NBUF]
    c = lax.axis_index("c")
    s = lax.axis_index("s")
    wid = s * NC + c
    n, d = acc_sh.shape
    pltpu.sync_copy(src_hbm.at[wid], src_v)
    pltpu.sync_copy(dst_hbm.at[wid], dst_v)

    nz = n // ZR
    nk = (nz + NS - 1) // NS
    nchunks = src_v.shape[0]

    for f, h2f_hbm in enumerate(h2_hbms):
        def fill_zeros(i, _):
            for g in range(d // 16):
                rows[0][i, pl.ds(g * 16, 16)] = jnp.zeros((16,),
                                                          jnp.float32)
            return 0

        lax.fori_loop(0, ZR, fill_zeros, 0)

        def zero_acc(k, _):
            j = s + k * NS

            @pl.when(j < nz)
            def _():
                pltpu.sync_copy(rows[0], acc_sh.at[pl.ds(j * ZR, ZR)])

            return 0

        lax.fori_loop(0, nk, zero_acc, 0)
        plsc.subcore_barrier()

        for m in range(min(NBUF - 1, nchunks)):
            pltpu.async_copy(h2f_hbm.at[src_v.at[m]], rows[m], sems[m])

        nmain = nchunks // NBUF

        def body(k, _):
            m0 = k * NBUF
            for b in range(NBUF):
                m = m0 + b
                mp = m + NBUF - 1
                bp = (b + NBUF - 1) % NBUF

                @pl.when(mp < nchunks)
                def _():
                    pltpu.async_copy(h2f_hbm.at[src_v.at[mp]],
                                     rows[bp], sems[bp])

                pltpu.make_async_copy(
                    h2f_hbm.at[src_v.at[m]], rows[b], sems[b]).wait()
                pltpu.sync_copy(rows[b], acc_sh.at[dst_v.at[m]], add=True)
            return 0

        lax.fori_loop(0, nmain, body, 0)
        for m in range(nmain * NBUF, nchunks):
            b = m % NBUF
            pltpu.make_async_copy(
                h2f_hbm.at[src_v.at[m]], rows[b], sems[b]).wait()
            pltpu.sync_copy(rows[b], acc_sh.at[dst_v.at[m]], add=True)
        plsc.subcore_barrier()

        def write_out(k, _):
            j = s + k * NS

            @pl.when(j < nz)
            def _():
                pltpu.sync_copy(acc_sh.at[pl.ds(j * ZR, ZR)],
                                out_hbm.at[c, f, pl.ds(j * ZR, ZR)])

            return 0

        lax.fori_loop(0, nk, write_out, 0)
        plsc.subcore_barrier()


def _tcmm_body(x_ref, w_ref, h_ref):
    h_ref[...] = jnp.dot(x_ref[...], w_ref[...],
                         preferred_element_type=jnp.float32)


def _tcscale_body(h_ref, degt_ref, *h2_refs):
    deg = degt_ref[:, 0:1] + degt_ref[:, 1:2] + 1.0
    dinv = lax.rsqrt(deg)
    h2 = h_ref[...] * dinv
    for f, h2f_ref in enumerate(h2_refs):
        h2f_ref[...] = h2[:, f * DH:(f + 1) * DH]


def _tc2_body(acc_ref, *refs):
    h2_refs = refs[:NSPLIT]
    degt_ref, b_ref, out_ref = refs[NSPLIT:]
    deg = degt_ref[:, 0:1] + degt_ref[:, 1:2] + 1.0
    dinv = lax.rsqrt(deg)
    for f, h2f_ref in enumerate(h2_refs):
        total = acc_ref[0, f] + acc_ref[1, f] + h2f_ref[...]
        out_ref[:, f * DH:(f + 1) * DH] = (
            dinv * total + b_ref[:, f * DH:(f + 1) * DH])


def kernel(x, edge_index, W, b):
    n, d_in = x.shape
    d_out = W.shape[1]
    e = edge_index.shape[1]
    epw = e // NW
    nchunks = epw // CH
    src = edge_index[0].reshape(NW, nchunks, CH)
    dst = edge_index[1].reshape(NW, nchunks, CH)
    dst_deg = edge_index[1].reshape(NW, epw // DCH, DCH)

    mesh = plsc.VectorSubcoreMesh(**_MESH)

    deg_k = pl.kernel(
        _deg_body,
        out_type=jax.ShapeDtypeStruct((NC * n,), jnp.float32),
        mesh=mesh,
        scratch_types=[
            pltpu.VMEM((epw // DCH, DCH), jnp.int32),
            pltpu.VMEM((DCH,), jnp.float32),
            pltpu.VMEM((n,), jnp.float32),
            pltpu.VMEM_SHARED((n,), jnp.float32),
            pltpu.SemaphoreType.DMA,
        ],
    )
    BM = 1000
    grid = (n // BM,)

    h = pl.pallas_call(
        _tcmm_body,
        grid=grid,
        in_specs=[
            pl.BlockSpec((BM, d_in), lambda i: (i, 0)),
            pl.BlockSpec((d_in, d_out), lambda i: (0, 0)),
        ],
        out_specs=pl.BlockSpec((BM, d_out), lambda i: (i, 0)),
        out_shape=jax.ShapeDtypeStruct((n, d_out), jnp.float32),
    )(x, W)

    degp = deg_k(dst_deg).reshape(NC, n)
    degp_t = degp.T

    h2s = pl.pallas_call(
        _tcscale_body,
        grid=grid,
        in_specs=[
            pl.BlockSpec((BM, d_out), lambda i: (i, 0)),
            pl.BlockSpec((BM, NC), lambda i: (i, 0)),
        ],
        out_specs=[pl.BlockSpec((BM, DH), lambda i: (i, 0))] * NSPLIT,
        out_shape=[jax.ShapeDtypeStruct((n, DH), jnp.float32)] * NSPLIT,
    )(h, degp_t)

    agg_k = pl.kernel(
        _agg_body,
        out_type=jax.ShapeDtypeStruct((NC, NSPLIT, n, DH), jnp.float32),
        mesh=mesh,
        scratch_types=[
            pltpu.VMEM((nchunks, CH), jnp.int32),
            pltpu.VMEM((nchunks, CH), jnp.int32),
            *[pltpu.VMEM((CH, DH), jnp.float32) for _ in range(NBUF)],
            pltpu.VMEM_SHARED((n, DH), jnp.float32),
            *[pltpu.SemaphoreType.DMA for _ in range(NBUF)],
        ],
        compiler_params=pltpu.CompilerParams(use_tc_tiling_on_sc=False),
    )
    acc = agg_k(src, dst, *h2s)

    out = pl.pallas_call(
        _tc2_body,
        grid=grid,
        in_specs=(
            [pl.BlockSpec((NC, NSPLIT, BM, DH), lambda i: (0, 0, i, 0))]
            + [pl.BlockSpec((BM, DH), lambda i: (i, 0))] * NSPLIT
            + [pl.BlockSpec((BM, NC), lambda i: (i, 0)),
               pl.BlockSpec((1, d_out), lambda i: (0, 0))]
        ),
        out_specs=pl.BlockSpec((BM, d_out), lambda i: (i, 0)),
        out_shape=jax.ShapeDtypeStruct((n, d_out), jnp.float32),
    )(acc, *h2s, degp_t, b.reshape(1, d_out))
    return out


# speedup vs baseline: 1.0082x; 1.0082x over previous
import jax
import jax.numpy as jnp
from jax import lax
from jax.experimental import pallas as pl
from jax.experimental.pallas import tpu as pltpu
from jax.experimental.pallas import tpu_sc as plsc

NC, NS = 2, 16
NW = NC * NS
CH = 80
DCH = 80
NBUF = 3
ZR = 80
NSPLIT = 1
DH = 128 // NSPLIT

_MESH = dict(core_axis_name="c", subcore_axis_name="s")


def _deg_body(dst_hbm, out_hbm, dst_v, ones_v, zbuf_v, deg_sh, dsem):
    c = lax.axis_index("c")
    s = lax.axis_index("s")
    wid = s * NC + c
    n = deg_sh.shape[0]
    pltpu.sync_copy(dst_hbm.at[wid], dst_v)

    def fill_ones(i, _):
        ones_v[pl.ds(i * 16, 16)] = jnp.ones((16,), jnp.float32)
        return 0

    lax.fori_loop(0, DCH // 16, fill_ones, 0)

    def fill_zeros(i, _):
        zbuf_v[pl.ds(i * 16, 16)] = jnp.zeros((16,), jnp.float32)
        return 0

    lax.fori_loop(0, n // 16, fill_zeros, 0)

    @pl.when(s == 0)
    def _():
        pltpu.sync_copy(zbuf_v, deg_sh)

    plsc.subcore_barrier()

    nchunks = dst_v.shape[0]
    K = 25

    def body(g, _):
        for u in range(K):
            pltpu.async_copy(ones_v, deg_sh.at[dst_v.at[g * K + u]],
                             dsem, add=True)
        for u in range(K):
            pltpu.make_async_copy(
                ones_v, deg_sh.at[dst_v.at[g * K + u]], dsem).wait()
        return 0

    lax.fori_loop(0, nchunks // K, body, 0)
    for j in range((nchunks // K) * K, nchunks):
        pltpu.sync_copy(ones_v, deg_sh.at[dst_v.at[j]], add=True)
    plsc.subcore_barrier()

    @pl.when(s == 0)
    def _():
        pltpu.sync_copy(deg_sh, zbuf_v)
        pltpu.sync_copy(zbuf_v, out_hbm.at[pl.ds(c * n, n)])


def _agg_body(src_hbm, dst_hbm, *refs):
    h2_hbms = refs[:NSPLIT]
    out_hbm = refs[NSPLIT]
    src_v = refs[NSPLIT + 1]
    dst_v = refs[NSPLIT + 2]
    rows = refs[NSPLIT + 3:NSPLIT + 3 + NBUF]
    acc_sh = refs[NSPLIT + 3 + NBUF]
    sems = refs[NSPLIT + 4 + NBUF:NSPLIT + 4 + 2 * NBUF]
    c = lax.axis_index("c")
    s = lax.axis_index("s")
    wid = s * NC + c
    n, d = acc_sh.shape
    pltpu.sync_copy(src_hbm.at[wid], src_v)
    pltpu.sync_copy(dst_hbm.at[wid], dst_v)

    nz = n // ZR
    nk = (nz + NS - 1) // NS
    nchunks = src_v.shape[0]

    for f, h2f_hbm in enumerate(h2_hbms):
        def fill_zeros(i, _):
            for g in range(d // 16):
                rows[0][i, pl.ds(g * 16, 16)] = jnp.zeros((16,),
                                                          jnp.float32)
            return 0

        lax.fori_loop(0, ZR, fill_zeros, 0)

        def zero_acc(k, _):
            j = s + k * NS

            @pl.when(j < nz)
            def _():
                pltpu.sync_copy(rows[0], acc_sh.at[pl.ds(j * ZR, ZR)])

            return 0

        lax.fori_loop(0, nk, zero_acc, 0)
        plsc.subcore_barrier()

        for m in range(min(NBUF - 1, nchunks)):
            pltpu.async_copy(h2f_hbm.at[src_v.at[m]], rows[m], sems[m])

        nmain = nchunks // NBUF

        def body(k, _):
            m0 = k * NBUF
            for b in range(NBUF):
                m = m0 + b
                mp = m + NBUF - 1
                bp = (b + NBUF - 1) % NBUF

                @pl.when(mp < nchunks)
                def _():
                    pltpu.async_copy(h2f_hbm.at[src_v.at[mp]],
                                     rows[bp], sems[bp])

                pltpu.make_async_copy(
                    h2f_hbm.at[src_v.at[m]], rows[b], sems[b]).wait()
                pltpu.sync_copy(rows[b], acc_sh.at[dst_v.at[m]], add=True)
            return 0

        lax.fori_loop(0, nmain, body, 0)
        for m in range(nmain * NBUF, nchunks):
            b = m % NBUF
            pltpu.make_async_copy(
                h2f_hbm.at[src_v.at[m]], rows[b], sems[b]).wait()
            pltpu.sync_copy(rows[b], acc_sh.at[dst_v.at[m]], add=True)
        plsc.subcore_barrier()

        def write_out(k, _):
            j = s + k * NS

            @pl.when(j < nz)
            def _():
                pltpu.sync_copy(acc_sh.at[pl.ds(j * ZR, ZR)],
                                out_hbm.at[c, f, pl.ds(j * ZR, ZR)])

            return 0

        lax.fori_loop(0, nk, write_out, 0)
        plsc.subcore_barrier()


def _tc1_body(x_ref, w_ref, degt_ref, *h2_refs):
    deg = degt_ref[:, 0:1] + degt_ref[:, 1:2] + 1.0
    dinv = lax.rsqrt(deg)
    h = jnp.dot(x_ref[...], w_ref[...], preferred_element_type=jnp.float32)
    h2 = h * dinv
    for f, h2f_ref in enumerate(h2_refs):
        h2f_ref[...] = h2[:, f * DH:(f + 1) * DH]


def _tc2_body(acc_ref, *refs):
    h2_refs = refs[:NSPLIT]
    degt_ref, b_ref, out_ref = refs[NSPLIT:]
    deg = degt_ref[:, 0:1] + degt_ref[:, 1:2] + 1.0
    dinv = lax.rsqrt(deg)
    for f, h2f_ref in enumerate(h2_refs):
        total = acc_ref[0, f] + acc_ref[1, f] + h2f_ref[...]
        out_ref[:, f * DH:(f + 1) * DH] = (
            dinv * total + b_ref[:, f * DH:(f + 1) * DH])


def kernel(x, edge_index, W, b):
    n, d_in = x.shape
    d_out = W.shape[1]
    e = edge_index.shape[1]
    epw = e // NW
    nchunks = epw // CH
    src = edge_index[0].reshape(NW, nchunks, CH)
    dst = edge_index[1].reshape(NW, nchunks, CH)
    dst_deg = edge_index[1].reshape(NW, epw // DCH, DCH)

    mesh = plsc.VectorSubcoreMesh(**_MESH)

    deg_k = pl.kernel(
        _deg_body,
        out_type=jax.ShapeDtypeStruct((NC * n,), jnp.float32),
        mesh=mesh,
        scratch_types=[
            pltpu.VMEM((epw // DCH, DCH), jnp.int32),
            pltpu.VMEM((DCH,), jnp.float32),
            pltpu.VMEM((n,), jnp.float32),
            pltpu.VMEM_SHARED((n,), jnp.float32),
            pltpu.SemaphoreType.DMA,
        ],
    )
    degp = deg_k(dst_deg).reshape(NC, n)
    degp_t = degp.T

    BM = 1000
    grid = (n // BM,)

    h2s = pl.pallas_call(
        _tc1_body,
        grid=grid,
        in_specs=[
            pl.BlockSpec((BM, d_in), lambda i: (i, 0)),
            pl.BlockSpec((d_in, d_out), lambda i: (0, 0)),
            pl.BlockSpec((BM, NC), lambda i: (i, 0)),
        ],
        out_specs=[pl.BlockSpec((BM, DH), lambda i: (i, 0))] * NSPLIT,
        out_shape=[jax.ShapeDtypeStruct((n, DH), jnp.float32)] * NSPLIT,
    )(x, W, degp_t)

    agg_k = pl.kernel(
        _agg_body,
        out_type=jax.ShapeDtypeStruct((NC, NSPLIT, n, DH), jnp.float32),
        mesh=mesh,
        scratch_types=[
            pltpu.VMEM((nchunks, CH), jnp.int32),
            pltpu.VMEM((nchunks, CH), jnp.int32),
            *[pltpu.VMEM((CH, DH), jnp.float32) for _ in range(NBUF)],
            pltpu.VMEM_SHARED((n, DH), jnp.float32),
            *[pltpu.SemaphoreType.DMA for _ in range(NBUF)],
        ],
        compiler_params=pltpu.CompilerParams(use_tc_tiling_on_sc=False),
    )
    acc = agg_k(src, dst, *h2s)

    out = pl.pallas_call(
        _tc2_body,
        grid=grid,
        in_specs=(
            [pl.BlockSpec((NC, NSPLIT, BM, DH), lambda i: (0, 0, i, 0))]
            + [pl.BlockSpec((BM, DH), lambda i: (i, 0))] * NSPLIT
            + [pl.BlockSpec((BM, NC), lambda i: (i, 0)),
               pl.BlockSpec((1, d_out), lambda i: (0, 0))]
        ),
        out_specs=pl.BlockSpec((BM, d_out), lambda i: (i, 0)),
        out_shape=jax.ShapeDtypeStruct((n, d_out), jnp.float32),
    )(acc, *h2s, degp_t, b.reshape(1, d_out))
    return out


# speedup vs baseline: 1.0172x; 1.0089x over previous
import jax
import jax.numpy as jnp
from jax import lax
from jax.experimental import pallas as pl
from jax.experimental.pallas import tpu as pltpu
from jax.experimental.pallas import tpu_sc as plsc

NC, NS = 2, 16
NW = NC * NS
CH = 80
DCH = 80
NBUF = 3
ZR = 80
NSPLIT = 1
DH = 128 // NSPLIT

_MESH = dict(core_axis_name="c", subcore_axis_name="s")


def _deg_body(dst_hbm, out_hbm, dst_v, ones_v, zbuf_v, deg_sh, dsem):
    c = lax.axis_index("c")
    s = lax.axis_index("s")
    wid = s * NC + c
    n = deg_sh.shape[0]
    pltpu.sync_copy(dst_hbm.at[wid], dst_v)

    def fill_ones(i, _):
        ones_v[pl.ds(i * 16, 16)] = jnp.ones((16,), jnp.float32)
        return 0

    lax.fori_loop(0, DCH // 16, fill_ones, 0)

    def fill_zeros(i, _):
        zbuf_v[pl.ds(i * 16, 16)] = jnp.zeros((16,), jnp.float32)
        return 0

    lax.fori_loop(0, n // 16, fill_zeros, 0)

    @pl.when(s == 0)
    def _():
        pltpu.sync_copy(zbuf_v, deg_sh)

    plsc.subcore_barrier()

    nchunks = dst_v.shape[0]
    K = 25

    def body(g, _):
        for u in range(K):
            pltpu.async_copy(ones_v, deg_sh.at[dst_v.at[g * K + u]],
                             dsem, add=True)
        for u in range(K):
            pltpu.make_async_copy(
                ones_v, deg_sh.at[dst_v.at[g * K + u]], dsem).wait()
        return 0

    lax.fori_loop(0, nchunks // K, body, 0)
    for j in range((nchunks // K) * K, nchunks):
        pltpu.sync_copy(ones_v, deg_sh.at[dst_v.at[j]], add=True)
    plsc.subcore_barrier()

    @pl.when(s == 0)
    def _():
        pltpu.sync_copy(deg_sh, zbuf_v)
        pltpu.sync_copy(zbuf_v, out_hbm.at[pl.ds(c * n, n)])


def _agg_body(src_hbm, dst_hbm, *refs):
    h2_hbms = refs[:NSPLIT]
    out_hbm = refs[NSPLIT]
    src_v = refs[NSPLIT + 1]
    dst_v = refs[NSPLIT + 2]
    rows = refs[NSPLIT + 3:NSPLIT + 3 + NBUF]
    acc_sh = refs[NSPLIT + 3 + NBUF]
    sems = refs[NSPLIT + 4 + NBUF:NSPLIT + 4 + 2 * NBUF]
    c = lax.axis_index("c")
    s = lax.axis_index("s")
    wid = s * NC + c
    n, d = acc_sh.shape
    pltpu.sync_copy(src_hbm.at[wid], src_v)
    pltpu.sync_copy(dst_hbm.at[wid], dst_v)

    nz = n // ZR
    nk = (nz + NS - 1) // NS
    nchunks = src_v.shape[0]

    for f, h2f_hbm in enumerate(h2_hbms):
        for m in range(min(NBUF - 1, nchunks)):
            pltpu.async_copy(h2f_hbm.at[src_v.at[m]], rows[m], sems[m])

        def fill_zeros(i, _):
            for g in range(d // 16):
                rows[NBUF - 1][i, pl.ds(g * 16, 16)] = jnp.zeros(
                    (16,), jnp.float32)
            return 0

        lax.fori_loop(0, ZR, fill_zeros, 0)

        def zero_acc(k, _):
            j = s + k * NS

            @pl.when(j < nz)
            def _():
                pltpu.sync_copy(rows[NBUF - 1],
                                acc_sh.at[pl.ds(j * ZR, ZR)])

            return 0

        lax.fori_loop(0, nk, zero_acc, 0)
        plsc.subcore_barrier()

        nmain = nchunks // NBUF

        def body(k, _):
            m0 = k * NBUF
            for b in range(NBUF):
                m = m0 + b
                mp = m + NBUF - 1
                bp = (b + NBUF - 1) % NBUF

                @pl.when(mp < nchunks)
                def _():
                    pltpu.async_copy(h2f_hbm.at[src_v.at[mp]],
                                     rows[bp], sems[bp])

                pltpu.make_async_copy(
                    h2f_hbm.at[src_v.at[m]], rows[b], sems[b]).wait()
                pltpu.sync_copy(rows[b], acc_sh.at[dst_v.at[m]], add=True)
            return 0

        lax.fori_loop(0, nmain, body, 0)
        for m in range(nmain * NBUF, nchunks):
            b = m % NBUF
            pltpu.make_async_copy(
                h2f_hbm.at[src_v.at[m]], rows[b], sems[b]).wait()
            pltpu.sync_copy(rows[b], acc_sh.at[dst_v.at[m]], add=True)
        plsc.subcore_barrier()

        def write_out(k, _):
            j = s + k * NS

            @pl.when(j < nz)
            def _():
                pltpu.sync_copy(acc_sh.at[pl.ds(j * ZR, ZR)],
                                out_hbm.at[c, f, pl.ds(j * ZR, ZR)])

            return 0

        lax.fori_loop(0, nk, write_out, 0)
        plsc.subcore_barrier()


def _tc1_body(x_ref, w_ref, degt_ref, *h2_refs):
    deg = degt_ref[:, 0:1] + degt_ref[:, 1:2] + 1.0
    dinv = lax.rsqrt(deg)
    h = jnp.dot(x_ref[...], w_ref[...], preferred_element_type=jnp.float32)
    h2 = h * dinv
    for f, h2f_ref in enumerate(h2_refs):
        h2f_ref[...] = h2[:, f * DH:(f + 1) * DH]


def _tc2_body(acc_ref, *refs):
    h2_refs = refs[:NSPLIT]
    degt_ref, b_ref, out_ref = refs[NSPLIT:]
    deg = degt_ref[:, 0:1] + degt_ref[:, 1:2] + 1.0
    dinv = lax.rsqrt(deg)
    for f, h2f_ref in enumerate(h2_refs):
        total = acc_ref[0, f] + acc_ref[1, f] + h2f_ref[...]
        out_ref[:, f * DH:(f + 1) * DH] = (
            dinv * total + b_ref[:, f * DH:(f + 1) * DH])


def kernel(x, edge_index, W, b):
    n, d_in = x.shape
    d_out = W.shape[1]
    e = edge_index.shape[1]
    epw = e // NW
    nchunks = epw // CH
    src = edge_index[0].reshape(NW, nchunks, CH)
    dst = edge_index[1].reshape(NW, nchunks, CH)
    dst_deg = edge_index[1].reshape(NW, epw // DCH, DCH)

    mesh = plsc.VectorSubcoreMesh(**_MESH)

    deg_k = pl.kernel(
        _deg_body,
        out_type=jax.ShapeDtypeStruct((NC * n,), jnp.float32),
        mesh=mesh,
        scratch_types=[
            pltpu.VMEM((epw // DCH, DCH), jnp.int32),
            pltpu.VMEM((DCH,), jnp.float32),
            pltpu.VMEM((n,), jnp.float32),
            pltpu.VMEM_SHARED((n,), jnp.float32),
            pltpu.SemaphoreType.DMA,
        ],
    )
    degp = deg_k(dst_deg).reshape(NC, n)
    degp_t = degp.T

    BM = 1000
    grid = (n // BM,)

    h2s = pl.pallas_call(
        _tc1_body,
        grid=grid,
        in_specs=[
            pl.BlockSpec((BM, d_in), lambda i: (i, 0)),
            pl.BlockSpec((d_in, d_out), lambda i: (0, 0)),
            pl.BlockSpec((BM, NC), lambda i: (i, 0)),
        ],
        out_specs=[pl.BlockSpec((BM, DH), lambda i: (i, 0))] * NSPLIT,
        out_shape=[jax.ShapeDtypeStruct((n, DH), jnp.float32)] * NSPLIT,
    )(x, W, degp_t)

    agg_k = pl.kernel(
        _agg_body,
        out_type=jax.ShapeDtypeStruct((NC, NSPLIT, n, DH), jnp.float32),
        mesh=mesh,
        scratch_types=[
            pltpu.VMEM((nchunks, CH), jnp.int32),
            pltpu.VMEM((nchunks, CH), jnp.int32),
            *[pltpu.VMEM((CH, DH), jnp.float32) for _ in range(NBUF)],
            pltpu.VMEM_SHARED((n, DH), jnp.float32),
            *[pltpu.SemaphoreType.DMA for _ in range(NBUF)],
        ],
        compiler_params=pltpu.CompilerParams(use_tc_tiling_on_sc=False),
    )
    acc = agg_k(src, dst, *h2s)

    out = pl.pallas_call(
        _tc2_body,
        grid=grid,
        in_specs=(
            [pl.BlockSpec((NC, NSPLIT, BM, DH), lambda i: (0, 0, i, 0))]
            + [pl.BlockSpec((BM, DH), lambda i: (i, 0))] * NSPLIT
            + [pl.BlockSpec((BM, NC), lambda i: (i, 0)),
               pl.BlockSpec((1, d_out), lambda i: (0, 0))]
        ),
        out_specs=pl.BlockSpec((BM, d_out), lambda i: (i, 0)),
        out_shape=jax.ShapeDtypeStruct((n, d_out), jnp.float32),
    )(acc, *h2s, degp_t, b.reshape(1, d_out))
    return out
